# baseline (speedup 1.0000x reference)
import functools

import jax
import jax.numpy as jnp
from jax import lax
from jax.experimental import pallas as pl
from jax.experimental.pallas import tpu as pltpu
from jax.experimental.pallas import tpu_sc as plsc

_NC = 2
_NS = 16
_NW = _NC * _NS
_L = 16
_BLK = 256
_EPAD = 16


def _make_dispatch(N, DIN, Nk, E, P, K):
    PP = P // _NW
    n_chunk = PP // _L
    GB = 64
    n_g = PP // GB
    QCH = 96
    n_q = PP // QCH
    mesh = plsc.VectorSubcoreMesh(
        core_axis_name="c", subcore_axis_name="s",
        num_cores=_NC, num_subcores=_NS)

    @functools.partial(
        pl.kernel,
        out_type=(jax.ShapeDtypeStruct((P, DIN), jnp.float32),
                  jax.ShapeDtypeStruct((Nk + _L,), jnp.int32)),
        mesh=mesh,
        compiler_params=pltpu.CompilerParams(needs_layout_passes=False),
        scratch_types=[
            pltpu.VMEM((Nk,), jnp.int32),
            pltpu.VMEM((_EPAD,), jnp.int32),
            pltpu.VMEM((_EPAD,), jnp.int32),
            pltpu.VMEM((_EPAD,), jnp.int32),
            [pltpu.VMEM((GB,), jnp.int32) for _ in range(n_g)],
            [pltpu.VMEM((GB, DIN), jnp.float32) for _ in range(2)],
            pltpu.VMEM((n_q, QCH), jnp.int32),
            pltpu.VMEM((n_q, QCH), jnp.int32),
            pltpu.SemaphoreType.DMA,
            pltpu.SemaphoreType.DMA,
            pltpu.SemaphoreType.DMA,
        ],
    )
    def dispatch(ssi_hbm, ps_hbm, gs_hbm, ge_hbm, inp_hbm, x_hbm, inv_hbm,
                 ssi_v, ps_v, gs_v, ge_v, tok_vs, rows_vs, qi_v, qp_v,
                 gsem, ssem, qsem):
        wid = lax.axis_index("s") * _NC + lax.axis_index("c")
        base = wid * PP
        pltpu.sync_copy(ssi_hbm, ssi_v)
        pltpu.sync_copy(ps_hbm, ps_v)
        pltpu.sync_copy(gs_hbm, gs_v)
        pltpu.sync_copy(ge_hbm, ge_v)
        ps_all = ps_v[...]
        for j in range(n_chunk):
            p = base + j * _L + lax.iota(jnp.int32, _L)
            e = jnp.zeros((_L,), jnp.int32)
            for ei in range(1, E):
                e = e + (p >= ps_all[ei]).astype(jnp.int32)
            ps_g = plsc.load_gather(ps_v, [e])
            gs_g = plsc.load_gather(gs_v, [e])
            ge_g = plsc.load_gather(ge_v, [e])
            r = p - ps_g + gs_g
            valid = r < ge_g
            r = jnp.where(valid, r, 0)
            q = plsc.load_gather(ssi_v, [r])
            tok = q // K
            tok_vs[j // (GB // _L)][pl.ds((j % (GB // _L)) * _L, _L)] = tok
            qi = jnp.where(valid, (q % K) * N + tok, Nk)
            qi_v[j // (QCH // _L), pl.ds((j % (QCH // _L)) * _L, _L)] = qi
            qp_v[j // (QCH // _L), pl.ds((j % (QCH // _L)) * _L, _L)] = p
        qd = [pltpu.async_copy(qp_v.at[c], inv_hbm.at[qi_v.at[c]], qsem)
              for c in range(n_q)]
        gd = [None] * n_g
        sd = [None] * n_g
        gd[0] = pltpu.async_copy(inp_hbm.at[tok_vs[0]], rows_vs[0], gsem)
        for c in range(n_g):
            gd[c].wait()
            sd[c] = pltpu.async_copy(
                rows_vs[c % 2], x_hbm.at[pl.ds(base + c * GB, GB), :], ssem)
            if c + 1 < n_g:
                if c >= 1:
                    sd[c - 1].wait()
                gd[c + 1] = pltpu.async_copy(
                    inp_hbm.at[tok_vs[c + 1]], rows_vs[(c + 1) % 2], gsem)
        for c in range(max(0, n_g - 2), n_g):
            sd[c].wait()
        for d in qd:
            d.wait()

    return dispatch


def _make_matmul(NBP, DIN, DOUT):
    def body(be_ref, x_ref, w_ref, y_ref):
        del be_ref
        y_ref[...] = jnp.dot(x_ref[...], w_ref[0],
                             preferred_element_type=jnp.float32)

    grid_spec = pltpu.PrefetchScalarGridSpec(
        num_scalar_prefetch=1,
        grid=(NBP,),
        in_specs=[
            pl.BlockSpec((_BLK, DIN), lambda b, be: (b, 0)),
            pl.BlockSpec((1, DIN, DOUT), lambda b, be: (be[b], 0, 0)),
        ],
        out_specs=pl.BlockSpec((_BLK, DOUT), lambda b, be: (b, 0)),
    )
    return pl.pallas_call(
        body, grid_spec=grid_spec,
        out_shape=jax.ShapeDtypeStruct((NBP * _BLK, DOUT), jnp.float32))


def _make_combine(N, DOUT, Nk, E, K):
    TPT = N // _NW
    TCK = 16
    n_chunk = TPT // TCK
    QC = TCK * K
    mesh = plsc.VectorSubcoreMesh(
        core_axis_name="c", subcore_axis_name="s",
        num_cores=_NC, num_subcores=_NS)

    @functools.partial(
        pl.kernel,
        out_type=jax.ShapeDtypeStruct((N, DOUT), jnp.float32),
        mesh=mesh,
        compiler_params=pltpu.CompilerParams(needs_layout_passes=False),
        scratch_types=[
            [pltpu.VMEM((QC,), jnp.int32) for _ in range(2)],
            pltpu.VMEM((TPT * K,), jnp.float32),
            [pltpu.VMEM((QC, DOUT), jnp.float32) for _ in range(2)],
            [pltpu.VMEM((TCK, DOUT), jnp.float32) for _ in range(2)],
            pltpu.SemaphoreType.DMA,
            pltpu.SemaphoreType.DMA,
        ],
    )
    def combine(inv_hbm, gates_hbm, y_hbm, out_hbm,
                idx_vs, g_v, y_vs, o_vs, gsem, ssem):
        wid = lax.axis_index("s") * _NC + lax.axis_index("c")
        tb0 = wid * TPT
        pltpu.sync_copy(gates_hbm.at[pl.ds(tb0 * K, TPT * K)], g_v)

        def fill_idx(c):
            for s in range(K):
                pltpu.sync_copy(
                    inv_hbm.at[pl.ds(s * N + tb0 + c * TCK, TCK)],
                    idx_vs[c % 2].at[pl.ds(s * TCK, TCK)])

        fill_idx(0)
        gd = [None] * n_chunk
        sd = [None] * n_chunk
        gd[0] = pltpu.async_copy(y_hbm.at[idx_vs[0]], y_vs[0], gsem)
        for c in range(n_chunk):
            if c + 1 < n_chunk:
                fill_idx(c + 1)
                gd[c + 1] = pltpu.async_copy(
                    y_hbm.at[idx_vs[(c + 1) % 2]], y_vs[(c + 1) % 2], gsem)
            if c >= 2:
                sd[c - 2].wait()
            gd[c].wait()
            ov = o_vs[c % 2]
            yv = y_vs[c % 2]

            @pl.loop(0, TCK)
            def _(t):
                gb = []
                for s in range(K):
                    gb.append(plsc.load_gather(
                        g_v, [jnp.full((_L,), (c * TCK + t) * K + s,
                                       jnp.int32)]))
                for lg in range(DOUT // _L):
                    sl = pl.ds(lg * _L, _L)
                    acc = jnp.zeros((_L,), jnp.float32)
                    for s in range(K):
                        acc = acc + gb[s] * yv[s * TCK + t, sl]
                    ov[t, sl] = acc

            sd[c] = pltpu.async_copy(
                ov, out_hbm.at[pl.ds(tb0 + c * TCK, TCK), :], ssem)
        for c in range(max(0, n_chunk - 2), n_chunk):
            sd[c].wait()

    return combine


def kernel(input, weight, k, sorted_expert_indices, sorted_scattered_indices,
           expert_offsets, gates):
    del sorted_expert_indices, k
    N, DIN = input.shape
    E, _, DOUT = weight.shape
    Nk = sorted_scattered_indices.shape[0]
    K = Nk // N
    NB = Nk // _BLK
    NBP = NB + E
    P = NBP * _BLK

    offs = expert_offsets.astype(jnp.int32)
    gstart = jnp.concatenate([jnp.zeros((1,), jnp.int32), offs[:-1]])
    gend = offs
    sizes = gend - gstart
    padded = ((sizes + _BLK - 1) // _BLK) * _BLK
    nblk = padded // _BLK
    cumblk = jnp.cumsum(nblk).astype(jnp.int32)
    pstart = jnp.concatenate(
        [jnp.zeros((1,), jnp.int32), jnp.cumsum(padded)[:-1].astype(jnp.int32)])
    block_expert = jnp.minimum(
        jnp.searchsorted(cumblk, jnp.arange(NBP, dtype=jnp.int32),
                         side="right"),
        E - 1).astype(jnp.int32)

    def pad16(a):
        return jnp.pad(a, (0, _EPAD - E), mode="edge")

    ssi = sorted_scattered_indices.astype(jnp.int32)
    gates_flat = gates.reshape(-1).astype(jnp.float32)

    x_padded, inv = _make_dispatch(N, DIN, Nk, E, P, K)(
        ssi, pad16(pstart), pad16(gstart), pad16(gend), input)
    y = _make_matmul(NBP, DIN, DOUT)(block_expert, x_padded, weight)
    out = _make_combine(N, DOUT, Nk, E, K)(inv, gates_flat, y)
    return out


# speedup vs baseline: 1.8942x; 1.8942x over previous
import functools

import jax
import jax.numpy as jnp
from jax import lax
from jax.experimental import pallas as pl
from jax.experimental.pallas import tpu as pltpu
from jax.experimental.pallas import tpu_sc as plsc

_NC = 2
_NS = 16
_NW = _NC * _NS
_L = 16
_BLK = 256
_EPAD = 16


def _make_dispatch(N, DIN, Nk, E, P, K):
    PP = P // _NW
    n_chunk = PP // _L
    GB = 64
    n_g = PP // GB
    mesh = plsc.VectorSubcoreMesh(
        core_axis_name="c", subcore_axis_name="s",
        num_cores=_NC, num_subcores=_NS)

    @functools.partial(
        pl.kernel,
        out_type=jax.ShapeDtypeStruct((P, DIN), jnp.float32),
        mesh=mesh,
        compiler_params=pltpu.CompilerParams(needs_layout_passes=False),
        scratch_types=[
            pltpu.VMEM((Nk,), jnp.int32),
            pltpu.VMEM((_EPAD,), jnp.int32),
            pltpu.VMEM((_EPAD,), jnp.int32),
            pltpu.VMEM((_EPAD,), jnp.int32),
            [pltpu.VMEM((GB,), jnp.int32) for _ in range(n_g)],
            [pltpu.VMEM((GB, DIN), jnp.float32) for _ in range(2)],
            pltpu.SemaphoreType.DMA,
            pltpu.SemaphoreType.DMA,
        ],
    )
    def dispatch(ssi_hbm, ps_hbm, gs_hbm, ge_hbm, inp_hbm, x_hbm,
                 ssi_v, ps_v, gs_v, ge_v, tok_vs, rows_vs, gsem, ssem):
        wid = lax.axis_index("s") * _NC + lax.axis_index("c")
        base = wid * PP
        pltpu.sync_copy(ssi_hbm, ssi_v)
        pltpu.sync_copy(ps_hbm, ps_v)
        pltpu.sync_copy(gs_hbm, gs_v)
        pltpu.sync_copy(ge_hbm, ge_v)
        ps_all = ps_v[...]
        for j in range(n_chunk):
            p = base + j * _L + lax.iota(jnp.int32, _L)
            e = jnp.zeros((_L,), jnp.int32)
            for ei in range(1, E):
                e = e + (p >= ps_all[ei]).astype(jnp.int32)
            ps_g = plsc.load_gather(ps_v, [e])
            gs_g = plsc.load_gather(gs_v, [e])
            ge_g = plsc.load_gather(ge_v, [e])
            r = p - ps_g + gs_g
            valid = r < ge_g
            r = jnp.where(valid, r, 0)
            q = plsc.load_gather(ssi_v, [r])
            tok = q // K
            tok_vs[j // (GB // _L)][pl.ds((j % (GB // _L)) * _L, _L)] = tok
        gd = [None] * n_g
        sd = [None] * n_g
        gd[0] = pltpu.async_copy(inp_hbm.at[tok_vs[0]], rows_vs[0], gsem)
        for c in range(n_g):
            gd[c].wait()
            sd[c] = pltpu.async_copy(
                rows_vs[c % 2], x_hbm.at[pl.ds(base + c * GB, GB), :], ssem)
            if c + 1 < n_g:
                if c >= 1:
                    sd[c - 1].wait()
                gd[c + 1] = pltpu.async_copy(
                    inp_hbm.at[tok_vs[c + 1]], rows_vs[(c + 1) % 2], gsem)
        for c in range(max(0, n_g - 2), n_g):
            sd[c].wait()

    return dispatch


def _make_matmul(NBP, DIN, DOUT):
    def body(be_ref, x_ref, w_ref, y_ref):
        del be_ref
        y_ref[...] = jnp.dot(x_ref[...], w_ref[0],
                             preferred_element_type=jnp.float32)

    grid_spec = pltpu.PrefetchScalarGridSpec(
        num_scalar_prefetch=1,
        grid=(NBP,),
        in_specs=[
            pl.BlockSpec((_BLK, DIN), lambda b, be: (b, 0)),
            pl.BlockSpec((1, DIN, DOUT), lambda b, be: (be[b], 0, 0)),
        ],
        out_specs=pl.BlockSpec((_BLK, DOUT), lambda b, be: (b, 0)),
    )
    return pl.pallas_call(
        body, grid_spec=grid_spec,
        out_shape=jax.ShapeDtypeStruct((NBP * _BLK, DOUT), jnp.float32))


def _make_combine(N, DOUT, Nk, E, K):
    TPT = N // _NW
    TCK = 16
    n_chunk = TPT // TCK
    QC = TCK * K
    mesh = plsc.VectorSubcoreMesh(
        core_axis_name="c", subcore_axis_name="s",
        num_cores=_NC, num_subcores=_NS)

    QT = TPT * K

    @functools.partial(
        pl.kernel,
        out_type=jax.ShapeDtypeStruct((N, DOUT), jnp.float32),
        mesh=mesh,
        compiler_params=pltpu.CompilerParams(needs_layout_passes=False),
        scratch_types=[
            pltpu.VMEM((Nk,), jnp.int32),
            pltpu.VMEM((_EPAD,), jnp.int32),
            pltpu.VMEM((_EPAD,), jnp.int32),
            pltpu.VMEM((_EPAD,), jnp.int32),
            pltpu.VMEM((QT,), jnp.int32),
            [pltpu.VMEM((QC,), jnp.int32) for _ in range(2)],
            pltpu.VMEM((QT,), jnp.float32),
            [pltpu.VMEM((QC, DOUT), jnp.float32) for _ in range(2)],
            [pltpu.VMEM((TCK, DOUT), jnp.float32) for _ in range(2)],
            pltpu.SemaphoreType.DMA,
            pltpu.SemaphoreType.DMA,
        ],
    )
    def combine(ssi_hbm, ps_hbm, gs_hbm, ge_hbm, gates_hbm, y_hbm, out_hbm,
                ssi_v, ps_v, gs_v, ge_v, r_v, idx_vs, g_v, y_vs, o_vs,
                gsem, ssem):
        wid = lax.axis_index("s") * _NC + lax.axis_index("c")
        tb0 = wid * TPT
        qlo = wid * QT
        pltpu.sync_copy(ssi_hbm, ssi_v)
        pltpu.sync_copy(ps_hbm, ps_v)
        pltpu.sync_copy(gs_hbm, gs_v)
        pltpu.sync_copy(ge_hbm, ge_v)
        pltpu.sync_copy(gates_hbm.at[pl.ds(qlo, QT)], g_v)

        @pl.loop(0, Nk // _L, unroll=4)
        def _(j):
            rr = j * _L + lax.iota(jnp.int32, _L)
            qv = ssi_v[pl.ds(j * _L, _L)]
            lq = qv - qlo
            m = (lq >= 0) & (lq < QT)
            plsc.store_scatter(r_v, [jnp.where(m, lq, 0)], rr, mask=m)

        ge_all = ge_v[...]

        def compute_idx(c):
            for jj in range(QC // _L):
                r = r_v[pl.ds(c * QC + jj * _L, _L)]
                e = jnp.zeros((_L,), jnp.int32)
                for ei in range(E - 1):
                    e = e + (r >= ge_all[ei]).astype(jnp.int32)
                ps_g = plsc.load_gather(ps_v, [e])
                gs_g = plsc.load_gather(gs_v, [e])
                idx_vs[c % 2][pl.ds(jj * _L, _L)] = r - gs_g + ps_g

        compute_idx(0)
        gd = [None] * n_chunk
        sd = [None] * n_chunk
        gd[0] = pltpu.async_copy(y_hbm.at[idx_vs[0]], y_vs[0], gsem)
        for c in range(n_chunk):
            if c + 1 < n_chunk:
                compute_idx(c + 1)
                gd[c + 1] = pltpu.async_copy(
                    y_hbm.at[idx_vs[(c + 1) % 2]], y_vs[(c + 1) % 2], gsem)
            if c >= 2:
                sd[c - 2].wait()
            gd[c].wait()
            ov = o_vs[c % 2]
            yv = y_vs[c % 2]

            @pl.loop(0, TCK)
            def _(t):
                gb = []
                for s in range(K):
                    gb.append(plsc.load_gather(
                        g_v, [jnp.full((_L,), (c * TCK + t) * K + s,
                                       jnp.int32)]))
                for lg in range(DOUT // _L):
                    sl = pl.ds(lg * _L, _L)
                    acc = jnp.zeros((_L,), jnp.float32)
                    for s in range(K):
                        acc = acc + gb[s] * yv[t * K + s, sl]
                    ov[t, sl] = acc

            sd[c] = pltpu.async_copy(
                ov, out_hbm.at[pl.ds(tb0 + c * TCK, TCK), :], ssem)
        for c in range(max(0, n_chunk - 2), n_chunk):
            sd[c].wait()

    return combine


def kernel(input, weight, k, sorted_expert_indices, sorted_scattered_indices,
           expert_offsets, gates):
    del sorted_expert_indices, k
    N, DIN = input.shape
    E, _, DOUT = weight.shape
    Nk = sorted_scattered_indices.shape[0]
    K = Nk // N
    NB = Nk // _BLK
    NBP = NB + E
    P = NBP * _BLK

    offs = expert_offsets.astype(jnp.int32)
    gstart = jnp.concatenate([jnp.zeros((1,), jnp.int32), offs[:-1]])
    gend = offs
    sizes = gend - gstart
    padded = ((sizes + _BLK - 1) // _BLK) * _BLK
    nblk = padded // _BLK
    cumblk = jnp.cumsum(nblk).astype(jnp.int32)
    pstart = jnp.concatenate(
        [jnp.zeros((1,), jnp.int32), jnp.cumsum(padded)[:-1].astype(jnp.int32)])
    block_expert = jnp.minimum(
        jnp.searchsorted(cumblk, jnp.arange(NBP, dtype=jnp.int32),
                         side="right"),
        E - 1).astype(jnp.int32)

    def pad16(a):
        return jnp.pad(a, (0, _EPAD - E), mode="edge")

    ssi = sorted_scattered_indices.astype(jnp.int32)
    gates_flat = gates.reshape(-1).astype(jnp.float32)

    x_padded = _make_dispatch(N, DIN, Nk, E, P, K)(
        ssi, pad16(pstart), pad16(gstart), pad16(gend), input)
    y = _make_matmul(NBP, DIN, DOUT)(block_expert, x_padded, weight)
    out = _make_combine(N, DOUT, Nk, E, K)(
        ssi, pad16(pstart), pad16(gstart), pad16(gend), gates_flat, y)
    return out


# speedup vs baseline: 3.8402x; 2.0273x over previous
import functools

import jax
import jax.numpy as jnp
from jax import lax
from jax.experimental import pallas as pl
from jax.experimental.pallas import tpu as pltpu
from jax.experimental.pallas import tpu_sc as plsc

_NC = 2
_NS = 16
_NW = _NC * _NS
_L = 16
_BLK = 256
_EPAD = 16


def _make_dispatch(N, DIN, Nk, E, P, K):
    TPT = N // _NW
    QT = TPT * K
    mesh = plsc.VectorSubcoreMesh(
        core_axis_name="c", subcore_axis_name="s",
        num_cores=_NC, num_subcores=_NS)

    @functools.partial(
        pl.kernel,
        out_type=jax.ShapeDtypeStruct((P, DIN), jnp.float32),
        mesh=mesh,
        compiler_params=pltpu.CompilerParams(needs_layout_passes=False),
        scratch_types=[
            pltpu.VMEM((Nk,), jnp.int32),
            pltpu.VMEM((_EPAD,), jnp.int32),
            pltpu.VMEM((_EPAD,), jnp.int32),
            pltpu.VMEM((_EPAD,), jnp.int32),
            pltpu.VMEM((QT,), jnp.int32),
            [pltpu.VMEM((TPT,), jnp.int32) for _ in range(K)],
            pltpu.VMEM((TPT, DIN), jnp.float32),
            pltpu.SemaphoreType.DMA,
        ],
    )
    def dispatch(ssi_hbm, ps_hbm, gs_hbm, ge_hbm, inp_hbm, x_hbm,
                 ssi_v, ps_v, gs_v, ge_v, r_v, idx_vs, rows_v, sem):
        wid = lax.axis_index("s") * _NC + lax.axis_index("c")
        tb0 = wid * TPT
        qlo = wid * QT
        rd = pltpu.async_copy(
            inp_hbm.at[pl.ds(tb0, TPT), :], rows_v, sem)
        pltpu.sync_copy(ssi_hbm, ssi_v)
        pltpu.sync_copy(ps_hbm, ps_v)
        pltpu.sync_copy(gs_hbm, gs_v)
        pltpu.sync_copy(ge_hbm, ge_v)

        @pl.loop(0, Nk // _L, unroll=4)
        def _(j):
            rr = j * _L + lax.iota(jnp.int32, _L)
            qv = ssi_v[pl.ds(j * _L, _L)]
            lq = qv - qlo
            m = (lq >= 0) & (lq < QT)
            dst = (lq % K) * TPT + lq // K
            plsc.store_scatter(r_v, [jnp.where(m, dst, 0)], rr, mask=m)

        ge_all = ge_v[...]
        for s in range(K):
            for jj in range(TPT // _L):
                r = r_v[pl.ds(s * TPT + jj * _L, _L)]
                e = jnp.zeros((_L,), jnp.int32)
                for ei in range(E - 1):
                    e = e + (r >= ge_all[ei]).astype(jnp.int32)
                ps_g = plsc.load_gather(ps_v, [e])
                gs_g = plsc.load_gather(gs_v, [e])
                idx_vs[s][pl.ds(jj * _L, _L)] = r - gs_g + ps_g
        rd.wait()
        sds = [pltpu.async_copy(rows_v, x_hbm.at[idx_vs[s]], sem)
               for s in range(K)]
        for d in sds:
            d.wait()

    return dispatch


def _make_matmul(NBP, DIN, DOUT):
    def body(be_ref, x_ref, w_ref, y_ref):
        del be_ref
        y_ref[...] = jnp.dot(x_ref[...], w_ref[0],
                             preferred_element_type=jnp.float32)

    grid_spec = pltpu.PrefetchScalarGridSpec(
        num_scalar_prefetch=1,
        grid=(NBP,),
        in_specs=[
            pl.BlockSpec((_BLK, DIN), lambda b, be: (b, 0)),
            pl.BlockSpec((1, DIN, DOUT), lambda b, be: (be[b], 0, 0)),
        ],
        out_specs=pl.BlockSpec((_BLK, DOUT), lambda b, be: (b, 0)),
    )
    return pl.pallas_call(
        body, grid_spec=grid_spec,
        out_shape=jax.ShapeDtypeStruct((NBP * _BLK, DOUT), jnp.float32))


def _make_combine(N, DOUT, Nk, E, K):
    TPT = N // _NW
    TCK = 16
    n_chunk = TPT // TCK
    QC = TCK * K
    mesh = plsc.VectorSubcoreMesh(
        core_axis_name="c", subcore_axis_name="s",
        num_cores=_NC, num_subcores=_NS)

    QT = TPT * K

    @functools.partial(
        pl.kernel,
        out_type=jax.ShapeDtypeStruct((N, DOUT), jnp.float32),
        mesh=mesh,
        compiler_params=pltpu.CompilerParams(needs_layout_passes=False),
        scratch_types=[
            pltpu.VMEM((Nk,), jnp.int32),
            pltpu.VMEM((_EPAD,), jnp.int32),
            pltpu.VMEM((_EPAD,), jnp.int32),
            pltpu.VMEM((_EPAD,), jnp.int32),
            pltpu.VMEM((QT,), jnp.int32),
            [pltpu.VMEM((QC,), jnp.int32) for _ in range(2)],
            pltpu.VMEM((QT,), jnp.float32),
            [pltpu.VMEM((QC, DOUT), jnp.float32) for _ in range(2)],
            [pltpu.VMEM((TCK, DOUT), jnp.float32) for _ in range(2)],
            pltpu.SemaphoreType.DMA,
            pltpu.SemaphoreType.DMA,
        ],
    )
    def combine(ssi_hbm, ps_hbm, gs_hbm, ge_hbm, gates_hbm, y_hbm, out_hbm,
                ssi_v, ps_v, gs_v, ge_v, r_v, idx_vs, g_v, y_vs, o_vs,
                gsem, ssem):
        wid = lax.axis_index("s") * _NC + lax.axis_index("c")
        tb0 = wid * TPT
        qlo = wid * QT
        pltpu.sync_copy(ssi_hbm, ssi_v)
        pltpu.sync_copy(ps_hbm, ps_v)
        pltpu.sync_copy(gs_hbm, gs_v)
        pltpu.sync_copy(ge_hbm, ge_v)
        pltpu.sync_copy(gates_hbm.at[pl.ds(qlo, QT)], g_v)

        @pl.loop(0, Nk // _L, unroll=4)
        def _(j):
            rr = j * _L + lax.iota(jnp.int32, _L)
            qv = ssi_v[pl.ds(j * _L, _L)]
            lq = qv - qlo
            m = (lq >= 0) & (lq < QT)
            plsc.store_scatter(r_v, [jnp.where(m, lq, 0)], rr, mask=m)

        ge_all = ge_v[...]

        def compute_idx(c):
            for jj in range(QC // _L):
                r = r_v[pl.ds(c * QC + jj * _L, _L)]
                e = jnp.zeros((_L,), jnp.int32)
                for ei in range(E - 1):
                    e = e + (r >= ge_all[ei]).astype(jnp.int32)
                ps_g = plsc.load_gather(ps_v, [e])
                gs_g = plsc.load_gather(gs_v, [e])
                idx_vs[c % 2][pl.ds(jj * _L, _L)] = r - gs_g + ps_g

        compute_idx(0)
        gd = [None] * n_chunk
        sd = [None] * n_chunk
        gd[0] = pltpu.async_copy(y_hbm.at[idx_vs[0]], y_vs[0], gsem)
        for c in range(n_chunk):
            if c + 1 < n_chunk:
                compute_idx(c + 1)
                gd[c + 1] = pltpu.async_copy(
                    y_hbm.at[idx_vs[(c + 1) % 2]], y_vs[(c + 1) % 2], gsem)
            if c >= 2:
                sd[c - 2].wait()
            gd[c].wait()
            ov = o_vs[c % 2]
            yv = y_vs[c % 2]

            @pl.loop(0, TCK)
            def _(t):
                gb = []
                for s in range(K):
                    gb.append(plsc.load_gather(
                        g_v, [jnp.full((_L,), (c * TCK + t) * K + s,
                                       jnp.int32)]))
                for lg in range(DOUT // _L):
                    sl = pl.ds(lg * _L, _L)
                    acc = jnp.zeros((_L,), jnp.float32)
                    for s in range(K):
                        acc = acc + gb[s] * yv[t * K + s, sl]
                    ov[t, sl] = acc

            sd[c] = pltpu.async_copy(
                ov, out_hbm.at[pl.ds(tb0 + c * TCK, TCK), :], ssem)
        for c in range(max(0, n_chunk - 2), n_chunk):
            sd[c].wait()

    return combine


def kernel(input, weight, k, sorted_expert_indices, sorted_scattered_indices,
           expert_offsets, gates):
    del sorted_expert_indices, k
    N, DIN = input.shape
    E, _, DOUT = weight.shape
    Nk = sorted_scattered_indices.shape[0]
    K = Nk // N
    NB = Nk // _BLK
    NBP = NB + E
    P = NBP * _BLK

    offs = expert_offsets.astype(jnp.int32)
    gstart = jnp.concatenate([jnp.zeros((1,), jnp.int32), offs[:-1]])
    gend = offs
    sizes = gend - gstart
    padded = ((sizes + _BLK - 1) // _BLK) * _BLK
    nblk = padded // _BLK
    cumblk = jnp.cumsum(nblk).astype(jnp.int32)
    pstart = jnp.concatenate(
        [jnp.zeros((1,), jnp.int32), jnp.cumsum(padded)[:-1].astype(jnp.int32)])
    block_expert = jnp.minimum(
        jnp.searchsorted(cumblk, jnp.arange(NBP, dtype=jnp.int32),
                         side="right"),
        E - 1).astype(jnp.int32)

    def pad16(a):
        return jnp.pad(a, (0, _EPAD - E), mode="edge")

    ssi = sorted_scattered_indices.astype(jnp.int32)
    gates_flat = gates.reshape(-1).astype(jnp.float32)

    x_padded = _make_dispatch(N, DIN, Nk, E, P, K)(
        ssi, pad16(pstart), pad16(gstart), pad16(gend), input)
    y = _make_matmul(NBP, DIN, DOUT)(block_expert, x_padded, weight)
    out = _make_combine(N, DOUT, Nk, E, K)(
        ssi, pad16(pstart), pad16(gstart), pad16(gend), gates_flat, y)
    return out
